# Initial kernel scaffold; baseline (speedup 1.0000x reference)
import jax
import jax.numpy as jnp
from jax.experimental import pallas as pl


def kernel(x, edge_index, g1, be1, W1, bW1, g2, be2, Wf1, bf1, Wf2, bf2):
    raise NotImplementedError("write your pallas kernel here")


# speedup vs baseline: 1.1004x; 1.1004x over previous
import functools

import jax
import jax.numpy as jnp
from jax import lax
from jax.experimental import pallas as pl
from jax.experimental.pallas import tpu as pltpu

N = 10000
DIM = 256
E = 160000
HID = DIM * 4
NEG = -1.0e9

ROWS_BLK = 1000


def _gelu_exact(x):
    return 0.5 * x * (1.0 + lax.erf(x * 0.7071067811865476))


def _ln(x, g, b, eps=1e-5):
    mu = jnp.mean(x, axis=-1, keepdims=True)
    var = jnp.mean((x - mu) ** 2, axis=-1, keepdims=True)
    return (x - mu) * lax.rsqrt(var + eps) * g + b


def _ln1_body(x_ref, g_ref, b_ref, o_ref):
    o_ref[...] = _ln(x_ref[...], g_ref[...], b_ref[...])


def _ln1(x2d, g1, be1):
    grid = (N // ROWS_BLK,)
    return pl.pallas_call(
        _ln1_body,
        grid=grid,
        in_specs=[
            pl.BlockSpec((ROWS_BLK, DIM), lambda i: (i, 0)),
            pl.BlockSpec((DIM,), lambda i: (0,)),
            pl.BlockSpec((DIM,), lambda i: (0,)),
        ],
        out_specs=pl.BlockSpec((ROWS_BLK, DIM), lambda i: (i, 0)),
        out_shape=jax.ShapeDtypeStruct((N, DIM), jnp.float32),
    )(x2d, g1, be1)


def _tail_body(aggr_ref, xn_ref, x_ref, W1_ref, bW1_ref, g2_ref, be2_ref,
               Wf1_ref, bf1_ref, Wf2_ref, bf2_ref, o_ref):
    aggr = aggr_ref[...]
    a = jnp.where(aggr == NEG, 0.0, aggr) - xn_ref[...]
    h = _gelu_exact(
        jnp.dot(a, W1_ref[...], preferred_element_type=jnp.float32) + bW1_ref[...])
    x1 = h + x_ref[...]
    xn2 = _ln(x1, g2_ref[...], be2_ref[...])
    hh = _gelu_exact(
        jnp.dot(xn2, Wf1_ref[...], preferred_element_type=jnp.float32) + bf1_ref[...])
    ff = jnp.dot(hh, Wf2_ref[...], preferred_element_type=jnp.float32) + bf2_ref[...]
    o_ref[...] = ff + x1


def _tail(aggr, xn, x2d, W1, bW1, g2, be2, Wf1, bf1, Wf2, bf2):
    grid = (N // ROWS_BLK,)
    row_spec = pl.BlockSpec((ROWS_BLK, DIM), lambda i: (i, 0))
    full = lambda shape: pl.BlockSpec(shape, lambda i: (0,) * len(shape))
    return pl.pallas_call(
        _tail_body,
        grid=grid,
        in_specs=[
            row_spec, row_spec, row_spec,
            full((DIM, DIM)), full((DIM,)), full((DIM,)), full((DIM,)),
            full((DIM, HID)), full((HID,)), full((HID, DIM)), full((DIM,)),
        ],
        out_specs=row_spec,
        out_shape=jax.ShapeDtypeStruct((N, DIM), jnp.float32),
    )(aggr, xn, x2d, W1, bW1, g2, be2, Wf1, bf1, Wf2, bf2)


def kernel(x, edge_index, g1, be1, W1, bW1, g2, be2, Wf1, bf1, Wf2, bf2):
    x2d = x.reshape(N, DIM)
    xn = _ln1(x2d, g1, be1)
    row, col = edge_index[0], edge_index[1]
    x_j = xn[col]
    aggr = jnp.full((N, DIM), NEG, dtype=jnp.float32).at[row].max(x_j)
    out = _tail(aggr, xn, x2d, W1, bW1, g2, be2, Wf1, bf1, Wf2, bf2)
    return out.reshape(1, N, DIM)
